# baseline (speedup 1.0000x reference)
import jax
import jax.numpy as jnp
from jax.experimental import pallas as pl
from jax.experimental.pallas import tpu as pltpu

_INDEX = 10


def _slice_body(x0_hbm, x1_hbm, o0_hbm, o1_hbm, s0, s1):
    c0 = pltpu.make_async_copy(x0_hbm.at[_INDEX], o0_hbm, s0)
    c1 = pltpu.make_async_copy(x1_hbm.at[_INDEX], o1_hbm, s1)
    c0.start()
    c1.start()
    c0.wait()
    c1.wait()


def kernel(x0, x1):
    B, S, D = x0.shape
    x0t = jnp.transpose(x0, (1, 2, 0))
    x1t = jnp.transpose(x1, (1, 2, 0))
    hbm = pl.BlockSpec(memory_space=pltpu.MemorySpace.HBM)
    o0t, o1t = pl.pallas_call(
        _slice_body,
        in_specs=[hbm, hbm],
        out_specs=[hbm, hbm],
        out_shape=[
            jax.ShapeDtypeStruct((D, B), x0.dtype),
            jax.ShapeDtypeStruct((D, B), x1.dtype),
        ],
        scratch_shapes=[
            pltpu.SemaphoreType.DMA,
            pltpu.SemaphoreType.DMA,
        ],
    )(x0t, x1t)
    return jnp.transpose(o0t, (1, 0)), jnp.transpose(o1t, (1, 0))


# speedup vs baseline: 10.1756x; 10.1756x over previous
import jax
import jax.numpy as jnp
from jax.experimental import pallas as pl
from jax.experimental.pallas import tpu as pltpu

_INDEX = 10
_BLOCK_B = 512


def _slice_body(x0_ref, x1_ref, o0_ref, o1_ref):
    o0_ref[...] = x0_ref[0]
    o1_ref[...] = x1_ref[0]


def kernel(x0, x1):
    B, S, D = x0.shape
    x0t = jnp.transpose(x0, (1, 2, 0))
    x1t = jnp.transpose(x1, (1, 2, 0))
    grid = (B // _BLOCK_B,)
    in_spec = pl.BlockSpec((1, D, _BLOCK_B), lambda i: (_INDEX, 0, i))
    out_spec = pl.BlockSpec((D, _BLOCK_B), lambda i: (0, i))
    o0t, o1t = pl.pallas_call(
        _slice_body,
        grid=grid,
        in_specs=[in_spec, in_spec],
        out_specs=[out_spec, out_spec],
        out_shape=[
            jax.ShapeDtypeStruct((D, B), x0.dtype),
            jax.ShapeDtypeStruct((D, B), x1.dtype),
        ],
        compiler_params=pltpu.CompilerParams(
            dimension_semantics=("parallel",),
        ),
    )(x0t, x1t)
    return jnp.transpose(o0t, (1, 0)), jnp.transpose(o1t, (1, 0))


# speedup vs baseline: 19.7842x; 1.9443x over previous
import jax
import jax.numpy as jnp
from jax.experimental import pallas as pl
from jax.experimental.pallas import tpu as pltpu

_INDEX = 10
_BLOCK_B = 2048


def _slice_body(x0_ref, x1_ref, o0_ref, o1_ref):
    o0_ref[...] = x0_ref[0]
    o1_ref[...] = x1_ref[0]


def kernel(x0, x1):
    B, S, D = x0.shape
    x0t = jnp.transpose(x0, (1, 2, 0))
    x1t = jnp.transpose(x1, (1, 2, 0))
    grid = (B // _BLOCK_B,)
    in_spec = pl.BlockSpec((1, D, _BLOCK_B), lambda i: (_INDEX, 0, i))
    out_spec = pl.BlockSpec((D, _BLOCK_B), lambda i: (0, i))
    o0t, o1t = pl.pallas_call(
        _slice_body,
        grid=grid,
        in_specs=[in_spec, in_spec],
        out_specs=[out_spec, out_spec],
        out_shape=[
            jax.ShapeDtypeStruct((D, B), x0.dtype),
            jax.ShapeDtypeStruct((D, B), x1.dtype),
        ],
        compiler_params=pltpu.CompilerParams(
            dimension_semantics=("parallel",),
        ),
    )(x0t, x1t)
    return jnp.transpose(o0t, (1, 0)), jnp.transpose(o1t, (1, 0))
